# baseline (speedup 1.0000x reference)
import functools

import jax
import jax.numpy as jnp
from jax import lax
from jax.experimental import pallas as pl
from jax.experimental.pallas import tpu as pltpu
from jax.experimental.pallas import tpu_sc as plsc

NC = 2
NS = 16
LANES = 16
CH = 128
NB = 4
NL = 3


def _make_kernel(n, d, e):
    dh = d // NC
    EP = e // NS
    FC = EP // CH
    TE = EP - FC * CH
    NQ = (FC - 4) // NB
    RB = 40
    step_rows = NS * RB
    ACC = ((n + step_rows - 1) // step_rows) * step_rows
    TPB = ACC // NS
    NRB = TPB // RB
    SH = max((n - 1).bit_length(), 1)
    MASK = (1 << SH) - 1

    assert 2 * SH <= 31 and e % NS == 0 and EP % 8 == 0
    assert n % RB == 0 and dh % LANES == 0
    assert FC >= 8 and (FC - 4) % NB == 0
    assert TE % LANES == 0 and TE > 0

    mesh = plsc.VectorSubcoreMesh(core_axis_name="c", subcore_axis_name="s")
    sds = jax.ShapeDtypeStruct

    @functools.partial(
        pl.kernel,
        mesh=mesh,
        compiler_params=pltpu.CompilerParams(use_tc_tiling_on_sc=False),
        out_type=(sds((NC, n, dh), jnp.float32),
                  sds((2, NC, n, dh), jnp.float32)),
        scratch_types=[
            pltpu.VMEM((EP,), jnp.int32),
            pltpu.VMEM((EP,), jnp.float32),
            pltpu.VMEM((16,), jnp.float32),
            pltpu.VMEM((RB, dh), jnp.float32),
            pltpu.VMEM_SHARED((ACC, dh), jnp.float32),
            [pltpu.VMEM((CH, dh), jnp.float32) for _ in range(NB)],
            [pltpu.VMEM((CH,), jnp.int32) for _ in range(NB)],
            [pltpu.VMEM((CH,), jnp.int32) for _ in range(NB)],
            pltpu.VMEM((TE, dh), jnp.float32),
            pltpu.VMEM((TE,), jnp.int32),
            pltpu.VMEM((TE,), jnp.int32),
            [pltpu.SemaphoreType.DMA for _ in range(NB)],
            [pltpu.SemaphoreType.DMA for _ in range(NB)],
            pltpu.SemaphoreType.DMA,
            pltpu.SemaphoreType.DMA,
        ],
    )
    def step(xs_hbm, pk_hbm, w_hbm, alpha_hbm,
             out_hbm, xb_hbm,
             pk_all, w_all, alpha_v, zbuf, acc, rows, gidx, sidx,
             rows_t, gidx_t, sidx_t, gsem, ssem, tsem, esem):
        c = lax.axis_index("c")
        s = lax.axis_index("s")

        def _unpack(ci, m, *, te=False):
            gi, si, ne = (gidx_t, sidx_t, TE) if te else (gidx[m], sidx[m], CH)
            for g in range(ne // LANES):
                p = pk_all[pl.ds(ci * CH + g * LANES, LANES)]
                gi[pl.ds(g * LANES, LANES)] = p & MASK
                si[pl.ds(g * LANES, LANES)] = p >> SH

        def _issue_g(sp, b, *, te=False):
            gi, rr, sem = (gidx_t, rows_t, tsem) if te else (gidx[b], rows[b], gsem[b])
            pltpu.async_copy(xb_hbm.at[sp, c].at[gi], rr, sem)

        def _wait_g(sp, b, *, te=False):
            gi, rr, sem = (gidx_t, rows_t, tsem) if te else (gidx[b], rows[b], gsem[b])
            pltpu.make_async_copy(xb_hbm.at[sp, c].at[gi], rr, sem).wait()

        def _issue_s(b):
            pltpu.async_copy(rows[b], acc.at[sidx[b]], ssem[b], add=True)

        def _wait_s(b):
            pltpu.make_async_copy(rows[b], acc.at[sidx[b]], ssem[b]).wait()

        def _scale(ci, b, *, te=False):
            rr, ne = (rows_t, TE) if te else (rows[b], CH)
            for g in range(ne // LANES):
                w16 = w_all[pl.ds(ci * CH + g * LANES, LANES)]
                for k in range(LANES):
                    wv = jnp.full((LANES,), w16[k], jnp.float32)
                    for j in range(dh // LANES):
                        sl = pl.ds(j * LANES, LANES)
                        rr[g * LANES + k, sl] = rr[g * LANES + k, sl] * wv

        pltpu.sync_copy(w_hbm.at[pl.ds(s * EP, EP)], w_all)
        pltpu.async_copy(pk_hbm.at[pl.ds(s * EP, EP)], pk_all, esem)
        pltpu.sync_copy(alpha_hbm, alpha_v)

        def _zrow(i, carry):
            for j in range(dh // LANES):
                zbuf[i, pl.ds(j * LANES, LANES)] = jnp.zeros((LANES,), jnp.float32)
            return carry
        lax.fori_loop(0, RB, _zrow, 0)

        alpha_all = alpha_v[pl.ds(0, LANES)]
        abuf, obuf = rows[0], rows[1]
        a0 = jnp.full((LANES,), alpha_all[0], jnp.float32)

        for k in range(NRB):
            r0 = s * TPB + k * RB

            @pl.when(r0 < n)
            def _():
                ab = abuf.at[pl.ds(0, RB)]
                ob = obuf.at[pl.ds(0, RB)]
                pltpu.sync_copy(xs_hbm.at[c].at[pl.ds(r0, RB)], ab)
                pltpu.async_copy(ab, xb_hbm.at[0, c].at[pl.ds(r0, RB)], esem)

                def _mix0(i3, cc):
                    for j in range(dh // LANES):
                        sl = pl.ds(j * LANES, LANES)
                        obuf[i3, sl] = a0 * abuf[i3, sl]
                    return cc
                lax.fori_loop(0, RB, _mix0, 0)
                pltpu.sync_copy(ob, out_hbm.at[c].at[pl.ds(r0, RB)])
                pltpu.make_async_copy(ab, xb_hbm.at[0, c].at[pl.ds(r0, RB)],
                                     esem).wait()
            pltpu.sync_copy(zbuf, acc.at[pl.ds(s * TPB + k * RB, RB)])
        pltpu.make_async_copy(pk_hbm.at[pl.ds(s * EP, EP)], pk_all, esem).wait()
        plsc.subcore_barrier()

        def _layer(l, carry):
            sp = (l + 1) % 2
            _unpack(0, 0)
            _unpack(1, 1)
            _issue_g(sp, 0)
            _issue_g(sp, 1)

            def _iter(ci, b, *, s_wait, pre):
                if s_wait:
                    _wait_s((b + 2) % NB)
                if pre:
                    _unpack(ci + 2, (b + 2) % NB)
                    _issue_g(sp, (b + 2) % NB)
                _wait_g(sp, b)
                _scale(ci, b)
                _issue_s(b)

            _iter(0, 0, s_wait=False, pre=True)
            _iter(1, 1, s_wait=False, pre=True)

            def _quad(q, cc):
                ci0 = q * NB + 2
                for o in range(NB):
                    _iter(ci0 + o, (2 + o) % NB, s_wait=True, pre=True)
                return cc
            lax.fori_loop(0, NQ, _quad, 0)

            _wait_s(0)
            _unpack(FC, 0, te=True)
            _issue_g(sp, 0, te=True)
            _wait_g(sp, 2)
            _scale(FC - 2, 2)
            _issue_s(2)
            _wait_s(1)
            _wait_g(sp, 3)
            _scale(FC - 1, 3)
            _issue_s(3)
            _wait_g(sp, 0, te=True)
            _scale(FC, 0, te=True)
            _wait_s(2)
            _wait_s(3)
            pltpu.sync_copy(rows_t, acc.at[sidx_t], add=True)
            plsc.subcore_barrier()

            a1 = jnp.full((LANES,), alpha_all[1], jnp.float32)
            a2 = jnp.full((LANES,), alpha_all[2], jnp.float32)
            a3 = jnp.full((LANES,), alpha_all[3], jnp.float32)
            lv = jnp.full((LANES,), l, jnp.int32)
            a_new = jnp.where(lv == 1, a1, jnp.where(lv == 2, a2, a3))
            for k in range(NRB):
                r0 = s * TPB + k * RB

                @pl.when(r0 < n)
                def _():
                    ab = abuf.at[pl.ds(0, RB)]
                    ob = obuf.at[pl.ds(0, RB)]
                    pltpu.sync_copy(acc.at[pl.ds(r0, RB)], ab)
                    pltpu.async_copy(ab, xb_hbm.at[l % 2, c].at[pl.ds(r0, RB)],
                                     esem)
                    pltpu.sync_copy(out_hbm.at[c].at[pl.ds(r0, RB)], ob)

                    def _mix(i3, cc):
                        for j in range(dh // LANES):
                            sl = pl.ds(j * LANES, LANES)
                            obuf[i3, sl] = obuf[i3, sl] + a_new * abuf[i3, sl]
                        return cc
                    lax.fori_loop(0, RB, _mix, 0)
                    pltpu.sync_copy(ob, out_hbm.at[c].at[pl.ds(r0, RB)])
                    pltpu.make_async_copy(
                        ab, xb_hbm.at[l % 2, c].at[pl.ds(r0, RB)], esem).wait()
                pltpu.sync_copy(zbuf, acc.at[pl.ds(s * TPB + k * RB, RB)])
            plsc.subcore_barrier()
            return carry
        lax.fori_loop(1, NL + 1, _layer, 0)

    return step


def kernel(edge_index, edge_values, emb_table, alpha):
    n, d = emb_table.shape
    e = edge_values.shape[0]
    src = edge_index[1].astype(jnp.int32)
    dst = edge_index[0].astype(jnp.int32)
    w = edge_values
    dh = d // NC
    sh = max((n - 1).bit_length(), 1)
    alpha_pad = jnp.zeros((16,), jnp.float32).at[: alpha.shape[0]].set(alpha)

    packed = jnp.left_shift(dst, sh) | src

    x = jnp.stack([emb_table[:, i * dh:(i + 1) * dh] for i in range(NC)])
    out, _ = _make_kernel(n, d, e)(x, packed, w, alpha_pad)

    out_full = jnp.concatenate([out[i] for i in range(NC)], axis=1)
    half = n // 2
    return out_full[:half], out_full[half:]


# speedup vs baseline: 1.0386x; 1.0386x over previous
import functools

import jax
import jax.numpy as jnp
from jax import lax
from jax.experimental import pallas as pl
from jax.experimental.pallas import tpu as pltpu
from jax.experimental.pallas import tpu_sc as plsc

NC = 2
NS = 16
LANES = 16
CH = 80
NB = 4
NL = 3


def _make_kernel(n, d, e):
    dh = d // NC
    EP = e // NS
    NCHUNK = EP // CH
    RB = 40
    step_rows = NS * RB
    ACC = ((n + step_rows - 1) // step_rows) * step_rows
    TPB = ACC // NS
    NRB = TPB // RB

    assert e % (NS * CH) == 0 and n % RB == 0 and dh % LANES == 0
    assert NCHUNK >= 8 and (NCHUNK - 6) % NB == 0

    mesh = plsc.VectorSubcoreMesh(core_axis_name="c", subcore_axis_name="s")
    sds = jax.ShapeDtypeStruct

    @functools.partial(
        pl.kernel,
        mesh=mesh,
        compiler_params=pltpu.CompilerParams(use_tc_tiling_on_sc=False),
        out_type=(sds((NC, n, dh), jnp.float32),
                  sds((2, NC, n, dh), jnp.float32)),
        scratch_types=[
            pltpu.VMEM((EP,), jnp.int32),
            pltpu.VMEM((EP,), jnp.int32),
            pltpu.VMEM((16,), jnp.float32),
            pltpu.VMEM((RB, dh), jnp.float32),
            pltpu.VMEM_SHARED((ACC, dh), jnp.float32),
            [pltpu.VMEM((CH, dh), jnp.float32) for _ in range(NB)],
            [pltpu.VMEM((CH,), jnp.int32) for _ in range(NB)],
            pltpu.VMEM((EP,), jnp.float32),
            [pltpu.SemaphoreType.DMA for _ in range(NB)],
            [pltpu.SemaphoreType.DMA for _ in range(NB)],
            pltpu.SemaphoreType.DMA,
        ],
    )
    def step(xs_hbm, src_hbm, dst_hbm, w_hbm, alpha_hbm,
             out_hbm, xb_hbm,
             src_all, dst_all, alpha_v, zbuf, acc, rows, sidx, w_all,
             gsem, ssem, esem):
        c = lax.axis_index("c")
        s = lax.axis_index("s")

        def _issue_g(sp, ci, b):
            pltpu.async_copy(
                xb_hbm.at[sp, c].at[src_all.at[pl.ds(ci * CH, CH)]],
                rows[b], gsem[b])

        def _wait_g(sp, ci, b):
            pltpu.make_async_copy(
                xb_hbm.at[sp, c].at[src_all.at[pl.ds(ci * CH, CH)]],
                rows[b], gsem[b]).wait()

        def _issue_s(b):
            pltpu.async_copy(rows[b], acc.at[sidx[b]], ssem[b], add=True)

        def _wait_s(b):
            pltpu.make_async_copy(rows[b], acc.at[sidx[b]], ssem[b]).wait()

        def _compute(ci, b):
            for g in range(CH // LANES):
                sidx[b][pl.ds(g * LANES, LANES)] = (
                    dst_all[pl.ds(ci * CH + g * LANES, LANES)])
                w16 = w_all[pl.ds(ci * CH + g * LANES, LANES)]
                for k in range(LANES):
                    wv = jnp.full((LANES,), w16[k], jnp.float32)
                    for j in range(dh // LANES):
                        sl = pl.ds(j * LANES, LANES)
                        r = rows[b]
                        r[g * LANES + k, sl] = r[g * LANES + k, sl] * wv

        pltpu.sync_copy(src_hbm.at[pl.ds(s * EP, EP)], src_all)
        pltpu.sync_copy(w_hbm.at[pl.ds(s * EP, EP)], w_all)
        pltpu.async_copy(dst_hbm.at[pl.ds(s * EP, EP)], dst_all, esem)
        pltpu.sync_copy(alpha_hbm, alpha_v)

        def _zrow(i, carry):
            for j in range(dh // LANES):
                zbuf[i, pl.ds(j * LANES, LANES)] = jnp.zeros((LANES,), jnp.float32)
            return carry
        lax.fori_loop(0, RB, _zrow, 0)

        alpha_all = alpha_v[pl.ds(0, LANES)]
        abuf, obuf = rows[0], rows[1]
        a0 = jnp.full((LANES,), alpha_all[0], jnp.float32)

        for k in range(NRB):
            r0 = s * TPB + k * RB

            @pl.when(r0 < n)
            def _():
                ab = abuf.at[pl.ds(0, RB)]
                ob = obuf.at[pl.ds(0, RB)]
                pltpu.sync_copy(xs_hbm.at[c].at[pl.ds(r0, RB)], ab)
                pltpu.async_copy(ab, xb_hbm.at[0, c].at[pl.ds(r0, RB)], esem)

                def _mix0(i3, cc):
                    for j in range(dh // LANES):
                        sl = pl.ds(j * LANES, LANES)
                        obuf[i3, sl] = a0 * abuf[i3, sl]
                    return cc
                lax.fori_loop(0, RB, _mix0, 0)
                pltpu.sync_copy(ob, out_hbm.at[c].at[pl.ds(r0, RB)])
                pltpu.make_async_copy(ab, xb_hbm.at[0, c].at[pl.ds(r0, RB)],
                                     esem).wait()
            pltpu.sync_copy(zbuf, acc.at[pl.ds(s * TPB + k * RB, RB)])
        pltpu.make_async_copy(dst_hbm.at[pl.ds(s * EP, EP)], dst_all, esem).wait()
        plsc.subcore_barrier()

        def _layer(l, carry):
            sp = (l + 1) % 2
            _issue_g(sp, 0, 0)
            _issue_g(sp, 1, 1)

            def _iter(ci, b, *, s_wait, g_issue):
                if s_wait:
                    _wait_s((b + 2) % NB)
                if g_issue:
                    _issue_g(sp, ci + 2, (b + 2) % NB)
                _wait_g(sp, ci, b)
                _compute(ci, b)
                _issue_s(b)

            _iter(0, 0, s_wait=False, g_issue=True)
            _iter(1, 1, s_wait=False, g_issue=True)

            NQ = (NCHUNK - 6) // NB

            def _quad(q, cc):
                ci0 = q * NB + 2
                for o in range(NB):
                    _iter(ci0 + o, (2 + o) % NB, s_wait=True, g_issue=True)
                return cc
            lax.fori_loop(0, NQ, _quad, 0)

            base_t = NQ * NB + 2
            for o in range(4):
                ci = base_t + o
                _iter(ci, (2 + o) % NB, s_wait=True, g_issue=(o < 2))
            _wait_s(0)
            _wait_s(1)
            plsc.subcore_barrier()

            a1 = jnp.full((LANES,), alpha_all[1], jnp.float32)
            a2 = jnp.full((LANES,), alpha_all[2], jnp.float32)
            a3 = jnp.full((LANES,), alpha_all[3], jnp.float32)
            lv = jnp.full((LANES,), l, jnp.int32)
            a_new = jnp.where(lv == 1, a1, jnp.where(lv == 2, a2, a3))
            for k in range(NRB):
                r0 = s * TPB + k * RB

                @pl.when(r0 < n)
                def _():
                    ab = abuf.at[pl.ds(0, RB)]
                    ob = obuf.at[pl.ds(0, RB)]
                    pltpu.sync_copy(acc.at[pl.ds(r0, RB)], ab)
                    pltpu.async_copy(ab, xb_hbm.at[l % 2, c].at[pl.ds(r0, RB)],
                                     esem)
                    pltpu.sync_copy(out_hbm.at[c].at[pl.ds(r0, RB)], ob)

                    def _mix(i3, cc):
                        for j in range(dh // LANES):
                            sl = pl.ds(j * LANES, LANES)
                            obuf[i3, sl] = obuf[i3, sl] + a_new * abuf[i3, sl]
                        return cc
                    lax.fori_loop(0, RB, _mix, 0)
                    pltpu.sync_copy(ob, out_hbm.at[c].at[pl.ds(r0, RB)])
                    pltpu.make_async_copy(
                        ab, xb_hbm.at[l % 2, c].at[pl.ds(r0, RB)], esem).wait()
                pltpu.sync_copy(zbuf, acc.at[pl.ds(s * TPB + k * RB, RB)])
            plsc.subcore_barrier()
            return carry
        lax.fori_loop(1, NL + 1, _layer, 0)

    return step


def kernel(edge_index, edge_values, emb_table, alpha):
    n, d = emb_table.shape
    e = edge_values.shape[0]
    src = edge_index[1]
    dst = edge_index[0]
    w = edge_values
    dh = d // NC
    alpha_pad = jnp.zeros((16,), jnp.float32).at[: alpha.shape[0]].set(alpha)

    x = jnp.stack([emb_table[:, i * dh:(i + 1) * dh] for i in range(NC)])
    out, _ = _make_kernel(n, d, e)(x, src, dst, w, alpha_pad)

    out_full = jnp.concatenate([out[i] for i in range(NC)], axis=1)
    half = n // 2
    return out_full[:half], out_full[half:]


# speedup vs baseline: 1.1193x; 1.0776x over previous
import functools

import jax
import jax.numpy as jnp
from jax import lax
from jax.experimental import pallas as pl
from jax.experimental.pallas import tpu as pltpu
from jax.experimental.pallas import tpu_sc as plsc

NC = 2
NS = 16
LANES = 16
CH = 80
NB = 4
NL = 3


def _make_kernel(n, d, e):
    dh = d // NC
    EP = e // NS
    NCHUNK = EP // CH
    RB = 80
    step_rows = NS * RB
    ACC = ((n + step_rows - 1) // step_rows) * step_rows
    TPB = ACC // NS
    NRB = TPB // RB

    assert e % (NS * CH) == 0 and n % RB == 0 and dh % LANES == 0
    assert NCHUNK >= 8 and (NCHUNK - 6) % NB == 0

    mesh = plsc.VectorSubcoreMesh(core_axis_name="c", subcore_axis_name="s")
    sds = jax.ShapeDtypeStruct

    @functools.partial(
        pl.kernel,
        mesh=mesh,
        compiler_params=pltpu.CompilerParams(use_tc_tiling_on_sc=False),
        out_type=(sds((NC, n, dh), jnp.float32),
                  sds((2, NC, n, dh), jnp.float32)),
        scratch_types=[
            pltpu.VMEM((EP,), jnp.int32),
            pltpu.VMEM((EP,), jnp.int32),
            pltpu.VMEM((16,), jnp.float32),
            pltpu.VMEM((RB, dh), jnp.float32),
            pltpu.VMEM_SHARED((ACC, dh), jnp.float32),
            [pltpu.VMEM((CH, dh), jnp.float32) for _ in range(NB)],
            [pltpu.VMEM((CH,), jnp.int32) for _ in range(NB)],
            [pltpu.VMEM((CH,), jnp.float32) for _ in range(NB)],
            [pltpu.SemaphoreType.DMA for _ in range(NB)],
            [pltpu.SemaphoreType.DMA for _ in range(NB)],
            [pltpu.SemaphoreType.DMA for _ in range(NB)],
            pltpu.SemaphoreType.DMA,
        ],
    )
    def step(xs_hbm, src_hbm, dst_hbm, w_hbm, alpha_hbm,
             out_hbm, xb_hbm,
             src_all, dst_all, alpha_v, zbuf, acc, rows, sidx, wbuf,
             gsem, ssem, wsem, esem):
        c = lax.axis_index("c")
        s = lax.axis_index("s")

        def _issue_w(ci, b):
            pltpu.async_copy(w_hbm.at[pl.ds(s * EP + ci * CH, CH)],
                             wbuf[b], wsem[b])

        def _wait_w(ci, b):
            pltpu.make_async_copy(w_hbm.at[pl.ds(s * EP + ci * CH, CH)],
                                  wbuf[b], wsem[b]).wait()

        def _issue_g(sp, ci, b):
            pltpu.async_copy(
                xb_hbm.at[sp, c].at[src_all.at[pl.ds(ci * CH, CH)]],
                rows[b], gsem[b])

        def _wait_g(sp, ci, b):
            pltpu.make_async_copy(
                xb_hbm.at[sp, c].at[src_all.at[pl.ds(ci * CH, CH)]],
                rows[b], gsem[b]).wait()

        def _issue_s(b):
            pltpu.async_copy(rows[b], acc.at[sidx[b]], ssem[b], add=True)

        def _wait_s(b):
            pltpu.make_async_copy(rows[b], acc.at[sidx[b]], ssem[b]).wait()

        def _compute(ci, b):
            for g in range(CH // LANES):
                sidx[b][pl.ds(g * LANES, LANES)] = (
                    dst_all[pl.ds(ci * CH + g * LANES, LANES)])
                w16 = wbuf[b][pl.ds(g * LANES, LANES)]
                for k in range(LANES):
                    wv = jnp.full((LANES,), w16[k], jnp.float32)
                    for j in range(dh // LANES):
                        sl = pl.ds(j * LANES, LANES)
                        r = rows[b]
                        r[g * LANES + k, sl] = r[g * LANES + k, sl] * wv

        pltpu.sync_copy(src_hbm.at[pl.ds(s * EP, EP)], src_all)
        pltpu.async_copy(dst_hbm.at[pl.ds(s * EP, EP)], dst_all, esem)
        pltpu.sync_copy(alpha_hbm, alpha_v)

        def _zrow(i, carry):
            for j in range(dh // LANES):
                zbuf[i, pl.ds(j * LANES, LANES)] = jnp.zeros((LANES,), jnp.float32)
            return carry
        lax.fori_loop(0, RB, _zrow, 0)

        alpha_all = alpha_v[pl.ds(0, LANES)]
        abuf, obuf = rows[0], rows[1]
        a0 = jnp.full((LANES,), alpha_all[0], jnp.float32)

        for k in range(NRB):
            r0 = s * TPB + k * RB

            @pl.when(r0 < n)
            def _():
                ab = abuf.at[pl.ds(0, RB)]
                ob = obuf.at[pl.ds(0, RB)]
                pltpu.sync_copy(xs_hbm.at[c].at[pl.ds(r0, RB)], ab)
                pltpu.async_copy(ab, xb_hbm.at[0, c].at[pl.ds(r0, RB)], esem)

                def _mix0(i3, cc):
                    for j in range(dh // LANES):
                        sl = pl.ds(j * LANES, LANES)
                        obuf[i3, sl] = a0 * abuf[i3, sl]
                    return cc
                lax.fori_loop(0, RB, _mix0, 0)
                pltpu.sync_copy(ob, out_hbm.at[c].at[pl.ds(r0, RB)])
                pltpu.make_async_copy(ab, xb_hbm.at[0, c].at[pl.ds(r0, RB)],
                                     esem).wait()
            pltpu.sync_copy(zbuf, acc.at[pl.ds(s * TPB + k * RB, RB)])
        pltpu.make_async_copy(dst_hbm.at[pl.ds(s * EP, EP)], dst_all, esem).wait()
        plsc.subcore_barrier()

        def _layer(l, carry):
            sp = (l + 1) % 2
            for b in range(NB):
                _issue_w(b, b)
            _issue_g(sp, 0, 0)
            _issue_g(sp, 1, 1)

            def _iter(ci, b, *, s_wait, g_issue, w_issue):
                if s_wait:
                    _wait_s((b + 2) % NB)
                if g_issue:
                    _issue_g(sp, ci + 2, (b + 2) % NB)
                _wait_w(ci, b)
                _wait_g(sp, ci, b)
                _compute(ci, b)
                _issue_s(b)
                if w_issue:
                    _issue_w(ci + NB, b)

            _iter(0, 0, s_wait=False, g_issue=True, w_issue=True)
            _iter(1, 1, s_wait=False, g_issue=True, w_issue=True)

            NQ = (NCHUNK - 6) // NB

            def _quad(q, cc):
                ci0 = q * NB + 2
                for o in range(NB):
                    _iter(ci0 + o, (2 + o) % NB, s_wait=True, g_issue=True,
                          w_issue=True)
                return cc
            lax.fori_loop(0, NQ, _quad, 0)

            base_t = NQ * NB + 2
            for o in range(4):
                ci = base_t + o
                _iter(ci, (2 + o) % NB, s_wait=True, g_issue=(o < 2),
                      w_issue=False)
            _wait_s(0)
            _wait_s(1)
            plsc.subcore_barrier()

            a1 = jnp.full((LANES,), alpha_all[1], jnp.float32)
            a2 = jnp.full((LANES,), alpha_all[2], jnp.float32)
            a3 = jnp.full((LANES,), alpha_all[3], jnp.float32)
            lv = jnp.full((LANES,), l, jnp.int32)
            a_new = jnp.where(lv == 1, a1, jnp.where(lv == 2, a2, a3))
            for k in range(NRB):
                r0 = s * TPB + k * RB

                @pl.when(r0 < n)
                def _():
                    ab = abuf.at[pl.ds(0, RB)]
                    ob = obuf.at[pl.ds(0, RB)]
                    pltpu.sync_copy(acc.at[pl.ds(r0, RB)], ab)
                    pltpu.async_copy(ab, xb_hbm.at[l % 2, c].at[pl.ds(r0, RB)],
                                     esem)
                    pltpu.sync_copy(out_hbm.at[c].at[pl.ds(r0, RB)], ob)

                    def _mix(i3, cc):
                        for j in range(dh // LANES):
                            sl = pl.ds(j * LANES, LANES)
                            obuf[i3, sl] = obuf[i3, sl] + a_new * abuf[i3, sl]
                        return cc
                    lax.fori_loop(0, RB, _mix, 0)
                    pltpu.sync_copy(ob, out_hbm.at[c].at[pl.ds(r0, RB)])
                    pltpu.make_async_copy(
                        ab, xb_hbm.at[l % 2, c].at[pl.ds(r0, RB)], esem).wait()
                pltpu.sync_copy(zbuf, acc.at[pl.ds(s * TPB + k * RB, RB)])
            plsc.subcore_barrier()
            return carry
        lax.fori_loop(1, NL + 1, _layer, 0)

    return step


def kernel(edge_index, edge_values, emb_table, alpha):
    n, d = emb_table.shape
    e = edge_values.shape[0]
    src = edge_index[1]
    dst = edge_index[0]
    w = edge_values
    dh = d // NC
    alpha_pad = jnp.zeros((16,), jnp.float32).at[: alpha.shape[0]].set(alpha)

    x = jnp.stack([emb_table[:, i * dh:(i + 1) * dh] for i in range(NC)])
    out, _ = _make_kernel(n, d, e)(x, src, dst, w, alpha_pad)

    out_full = jnp.concatenate([out[i] for i in range(NC)], axis=1)
    half = n // 2
    return out_full[:half], out_full[half:]
